# Initial kernel scaffold; baseline (speedup 1.0000x reference)
import jax
import jax.numpy as jnp
from jax.experimental import pallas as pl


def kernel(data, edge_index, depth, W_root, W_nbr, b, gamma, beta):
    raise NotImplementedError("write your pallas kernel here")


# speedup vs baseline: 5.5551x; 5.5551x over previous
import functools

import jax
import jax.numpy as jnp
import numpy as np
from jax import lax
from jax.experimental import pallas as pl
from jax.experimental.pallas import tpu as pltpu
from jax.experimental.pallas import tpu_sc as plsc

NUM_GROUPS = 4
EPS = 1e-5

_NC = 2
_NS = 16
_CHUNK = 80


def _sc_scatter_fn(N, C, E):
    n_workers = _NC * _NS
    assert E % (n_workers * _CHUNK) == 0, "edge count must split evenly"
    edges_per_tile = E // n_workers
    n_steps = edges_per_tile // _CHUNK
    assert N % 8 == 0
    rows_main = (N // (_NS * 8)) * 8
    rem = N - rows_main * _NS
    zrows = next(z for z in (256, 248, 240, 232, 224, 216, 208, 200, 192,
                             184, 176, 168, 160, 152, 144, 136, 128, 120,
                             112, 104, 96, 88, 80, 72, 64, 56, 48, 40, 32,
                             24, 16, 8)
                 if rows_main % z == 0)
    n_zcopies = rows_main // zrows
    assert rem % 8 == 0 and rem <= zrows

    mesh = plsc.VectorSubcoreMesh(core_axis_name="c", subcore_axis_name="s")

    @functools.partial(
        pl.kernel,
        out_type=jax.ShapeDtypeStruct((_NC, N, C), jnp.float32),
        mesh=mesh,
        scratch_types=[
            pltpu.VMEM((_CHUNK,), jnp.int32),
            pltpu.VMEM((_CHUNK,), jnp.int32),
            pltpu.VMEM((_CHUNK, C), jnp.float32),
            pltpu.VMEM((zrows, C), jnp.float32),
            pltpu.SemaphoreType.DMA,
            pltpu.VMEM_SHARED((N, C), jnp.float32),
        ],
    )
    def sc_kernel(data_hbm, src_hbm, dst_hbm, out_hbm,
                  idx_s, idx_d, rows, zbuf, sem, agg):
        c = lax.axis_index("c")
        s = lax.axis_index("s")

        zv = jnp.zeros((16,), jnp.float32)

        def zstore(i, carry):
            zbuf[i // (C // 16), pl.ds((i % (C // 16)) * 16, 16)] = zv
            return carry

        lax.fori_loop(0, zrows * (C // 16), zstore, 0)
        row0 = s * rows_main
        for j in range(n_zcopies):
            pltpu.sync_copy(zbuf, agg.at[pl.ds(row0 + j * zrows, zrows)])
        if rem:
            @pl.when(s == 0)
            def _():
                pltpu.sync_copy(zbuf.at[pl.ds(0, rem)],
                                agg.at[pl.ds(_NS * rows_main, rem)])
        plsc.subcore_barrier()

        base = (c * _NS + s) * edges_per_tile

        def step(k, carry):
            off = pl.multiple_of(base + k * _CHUNK, _CHUNK)
            pltpu.sync_copy(src_hbm.at[pl.ds(off, _CHUNK)], idx_s)
            pltpu.sync_copy(dst_hbm.at[pl.ds(off, _CHUNK)], idx_d)
            pltpu.async_copy(data_hbm.at[idx_s], rows, sem).wait()
            pltpu.sync_copy(rows, agg.at[idx_d], add=True)
            return carry

        lax.fori_loop(0, n_steps, step, 0)
        plsc.subcore_barrier()

        for j in range(n_zcopies):
            r = row0 + j * zrows
            pltpu.sync_copy(agg.at[pl.ds(r, zrows)], out_hbm.at[c, pl.ds(r, zrows)])
        if rem:
            @pl.when(s == 0)
            def _():
                r = _NS * rows_main
                pltpu.sync_copy(agg.at[pl.ds(r, rem)], out_hbm.at[c, pl.ds(r, rem)])

    return sc_kernel


def _tc_body(x_ref, p_ref, wr_ref, wn_ref, pm_ref, b_ref, g_ref, bt_ref, o_ref):
    x = x_ref[...]
    agg = p_ref[0] + p_ref[1]
    y = jnp.dot(x, wr_ref[...], preferred_element_type=jnp.float32)
    y = y + jnp.dot(agg, wn_ref[...], preferred_element_type=jnp.float32)
    y = y + b_ref[...]
    pm = pm_ref[...]
    m = jnp.dot(y, pm, preferred_element_type=jnp.float32)
    q = jnp.dot(y * y, pm, preferred_element_type=jnp.float32)
    yn = (y - m) * lax.rsqrt(q - m * m + EPS)
    o_ref[...] = jnp.maximum(yn * g_ref[...] + bt_ref[...], 0.0)


def kernel(data, edge_index, depth, W_root, W_nbr, b, gamma, beta):
    del depth
    N, C = data.shape
    E = edge_index.shape[1]
    src = edge_index[0]
    dst = edge_index[1]

    partials = _sc_scatter_fn(N, C, E)(data, src, dst)

    gsz = C // NUM_GROUPS
    pm = jnp.asarray(np.kron(np.eye(NUM_GROUPS, dtype=np.float32),
                             np.full((gsz, gsz), 1.0 / gsz, dtype=np.float32)))

    blk = 1000
    assert N % blk == 0
    out = pl.pallas_call(
        _tc_body,
        grid=(N // blk,),
        in_specs=[
            pl.BlockSpec((blk, C), lambda i: (i, 0)),
            pl.BlockSpec((_NC, blk, C), lambda i: (0, i, 0)),
            pl.BlockSpec((C, C), lambda i: (0, 0)),
            pl.BlockSpec((C, C), lambda i: (0, 0)),
            pl.BlockSpec((C, C), lambda i: (0, 0)),
            pl.BlockSpec((1, C), lambda i: (0, 0)),
            pl.BlockSpec((1, C), lambda i: (0, 0)),
            pl.BlockSpec((1, C), lambda i: (0, 0)),
        ],
        out_specs=pl.BlockSpec((blk, C), lambda i: (i, 0)),
        out_shape=jax.ShapeDtypeStruct((N, C), jnp.float32),
    )(data, partials, W_root, W_nbr, pm,
      b.reshape(1, C), gamma.reshape(1, C), beta.reshape(1, C))
    return out
